# Initial kernel scaffold; baseline (speedup 1.0000x reference)
import jax
import jax.numpy as jnp
from jax.experimental import pallas as pl


def kernel(global_feature, map_feature, factory_feature, unit_feature, location_feature, va_factory_act, va_move, va_transfer, va_pickup, va_dig, va_self_destruct, va_recharge, va_do_nothing, g_W, g_b, f_W, f_b, u_W, u_b, m_W, m_b, ld_W, ld_b, c_W, c_b):
    raise NotImplementedError("write your pallas kernel here")


# speedup vs baseline: 2.2385x; 2.2385x over previous
import functools

import jax
import jax.numpy as jnp
from jax import lax
from jax.experimental import pallas as pl
from jax.experimental.pallas import tpu as pltpu
from jax.experimental.pallas import tpu_sc as plsc

_B, _H, _W = 64, 48, 48
_HW = _H * _W
_MAX_GROUP = 1000
_PADG = 1024
_PAD = 2
_PH, _PW = _H + 2 * _PAD, _W + 2 * _PAD

_NC, _NS = 2, 16
_NW = _NC * _NS
_BPW = _B // _NW
_CHUNKS = _HW // 16


def _leaky(x):
    return jnp.where(x >= 0, x, 0.01 * x)


def _fill(buf, x):
    buf[...] = jnp.zeros((_PH, _PW), jnp.float32)
    buf[pl.ds(_PAD, _H), pl.ds(_PAD, _W)] = x


def _shift(buf, dy, dx):
    return buf[pl.ds(_PAD + dy, _H), pl.ds(_PAD + dx, _W)]


def _avg3(buf, x):
    _fill(buf, x)
    acc = None
    for dy in (-1, 0, 1):
        for dx in (-1, 0, 1):
            s = _shift(buf, dy, dx)
            acc = s if acc is None else acc + s
    return acc * (1.0 / 9.0)


def _dense_body(gf_ref, map_ref, fac_ref, unit_ref,
                va_fact_ref, va_move_ref, va_transfer_ref, va_pickup_ref,
                va_dig_ref, va_sd_ref, va_rech_ref, va_dn_ref,
                g_W, g_b, f_W, f_b, u_W, u_b, m_W, m_b, ld_W, ld_b, c_W, c_b,
                cvu_ref, cvf_ref, buf):
    b = pl.program_id(0)

    gf0 = gf_ref[b, 0]
    gf1 = gf_ref[b, 1]
    sg = c_b[0]
    for o in range(2):
        ge = _leaky(g_W[o, 0] * gf0 + g_W[o, 1] * gf1 + g_b[o])
        sg = sg + c_W[0, o] * ge

    me = []
    for o in range(2):
        acc = None
        for c in range(6):
            v = map_ref[0, c] * m_W[o, c]
            acc = v if acc is None else acc + v
        me.append(_leaky(acc + m_b[o]))

    q0 = _avg3(buf, me[0])
    q1 = _avg3(buf, me[1])
    z = [None] * 8
    for i, q in enumerate((q0, q1)):
        _fill(buf, q)
        for dy in range(-2, 3):
            for dx in range(-2, 3):
                s = _shift(buf, dy, dx)
                for o in range(8):
                    w = ld_W[o, i * 25 + (dy + 2) * 5 + (dx + 2)]
                    z[o] = s * w if z[o] is None else z[o] + s * w
    t = None
    for o in range(8):
        u = _leaky(z[o] + ld_b[o])
        v = c_W[0, 8 + o] * u
        t = v if t is None else t + v
    crit = _avg3(buf, t) + sg

    for (ref, wm, bm, nch, base) in ((fac_ref, f_W, f_b, 6, 2),
                                     (unit_ref, u_W, u_b, 4, 4)):
        for o in range(2):
            acc = None
            for c in range(nch):
                v = ref[0, c] * wm[o, c]
                acc = v if acc is None else acc + v
            crit = crit + c_W[0, base + o] * _leaky(acc + bm[o])
    for o in range(2):
        crit = crit + c_W[0, 6 + o] * me[o]

    def any_over(ref):
        acc = ref[0, 0]
        for k in range(1, ref.shape[1]):
            acc = jnp.logical_or(acc, ref[0, k])
        return acc

    fm = any_over(va_fact_ref)
    um = va_dn_ref[0]
    for ref in (va_move_ref, va_transfer_ref, va_pickup_ref,
                va_dig_ref, va_sd_ref, va_rech_ref):
        um = jnp.logical_or(um, any_over(ref))

    cvu_ref[0] = jnp.where(um, crit, 0.0)
    cvf_ref[0] = jnp.where(fm, crit, 0.0)


def _tc_dense(global_feature, map_feature, factory_feature, unit_feature,
              va_fact, va_move, va_transfer, va_pickup, va_dig, va_sd,
              va_rech, va_dn,
              g_W, g_b, f_W, f_b, u_W, u_b, m_W, m_b, ld_Wr, ld_b, c_W, c_b):
    def img_spec(nch):
        return pl.BlockSpec((1, nch, _H, _W), lambda b: (b, 0, 0, 0))

    smem = pl.BlockSpec(memory_space=pltpu.SMEM)
    in_specs = [
        smem,
        img_spec(6), img_spec(6), img_spec(4),
        img_spec(4), img_spec(10), img_spec(50), img_spec(10),
        img_spec(2), img_spec(2), img_spec(2),
        pl.BlockSpec((1, _H, _W), lambda b: (b, 0, 0)),
    ] + [smem] * 12
    out_specs = [pl.BlockSpec((1, _H, _W), lambda b: (b, 0, 0))] * 2
    return pl.pallas_call(
        _dense_body,
        grid=(_B,),
        in_specs=in_specs,
        out_specs=out_specs,
        out_shape=[jax.ShapeDtypeStruct((_B, _H, _W), jnp.float32)] * 2,
        scratch_shapes=[pltpu.VMEM((_PH, _PW), jnp.float32)],
    )(global_feature, map_feature, factory_feature, unit_feature,
      va_fact, va_move, va_transfer, va_pickup, va_dig, va_sd, va_rech, va_dn,
      g_W, g_b, f_W, f_b, u_W, u_b, m_W, m_b, ld_Wr, ld_b, c_W, c_b)


def _sc_scatter(ids_u, ids_f, cv_u, cv_f):
    mesh = plsc.VectorSubcoreMesh(core_axis_name="c", subcore_axis_name="s")

    @functools.partial(
        pl.kernel,
        out_type=jax.ShapeDtypeStruct((_B, _PADG), jnp.float32),
        mesh=mesh,
        scratch_types=[
            pltpu.VMEM((_HW,), jnp.int32),
            pltpu.VMEM((_HW,), jnp.int32),
            pltpu.VMEM((_HW,), jnp.float32),
            pltpu.VMEM((_HW,), jnp.float32),
            pltpu.VMEM((_PADG,), jnp.float32),
        ],
        compiler_params=pltpu.CompilerParams(needs_layout_passes=False),
    )
    def run(idsu_hbm, idsf_hbm, cvu_hbm, cvf_hbm, out_hbm,
            idsu_v, idsf_v, cvu_v, cvf_v, acc_v):
        wid = lax.axis_index("s") * _NC + lax.axis_index("c")
        for j in range(_BPW):
            b = wid * _BPW + j
            pltpu.sync_copy(idsu_hbm.at[b], idsu_v)
            pltpu.sync_copy(idsf_hbm.at[b], idsf_v)
            pltpu.sync_copy(cvu_hbm.at[b], cvu_v)
            pltpu.sync_copy(cvf_hbm.at[b], cvf_v)

            zeros16 = jnp.zeros((16,), jnp.float32)

            def zbody(i, _):
                acc_v[pl.ds(pl.multiple_of(i * 16, 16), 16)] = zeros16
                return 0

            lax.fori_loop(0, _PADG // 16, zbody, 0)

            def body(i, _):
                sl = pl.ds(pl.multiple_of(i * 16, 16), 16)
                plsc.addupdate_scatter(acc_v, [idsu_v[sl]], cvu_v[sl])
                plsc.addupdate_scatter(acc_v, [idsf_v[sl]], cvf_v[sl])
                return 0

            lax.fori_loop(0, _CHUNKS, body, 0)
            pltpu.sync_copy(acc_v, out_hbm.at[b])

    return run(ids_u, ids_f, cv_u, cv_f)


def kernel(global_feature, map_feature, factory_feature, unit_feature,
           location_feature, va_factory_act, va_move, va_transfer, va_pickup,
           va_dig, va_self_destruct, va_recharge, va_do_nothing,
           g_W, g_b, f_W, f_b, u_W, u_b, m_W, m_b, ld_W, ld_b, c_W, c_b):
    Bn = global_feature.shape[0]
    cv_u, cv_f = _tc_dense(
        global_feature, map_feature, factory_feature, unit_feature,
        va_factory_act,
        va_move.reshape(Bn, -1, _H, _W),
        va_transfer.reshape(Bn, -1, _H, _W),
        va_pickup.reshape(Bn, -1, _H, _W),
        va_dig, va_self_destruct, va_recharge, va_do_nothing,
        g_W, g_b, f_W, f_b, u_W, u_b, m_W, m_b,
        ld_W.reshape(8, 50), ld_b, c_W, c_b)
    ids = location_feature.astype(jnp.int32)
    ids_f = ids[:, 0].reshape(Bn, _HW)
    ids_u = ids[:, 1].reshape(Bn, _HW)
    out = _sc_scatter(ids_u, ids_f,
                      cv_u.reshape(Bn, _HW), cv_f.reshape(Bn, _HW))
    return out[:, :_MAX_GROUP]
